# baseline (speedup 1.0000x reference)
import functools

import jax
import jax.numpy as jnp
import numpy as np
from jax.experimental import pallas as pl
from jax.experimental.pallas import tpu as pltpu

_T = 32
_P = 3
_D = 256
_NG = 5
_J2 = 6
_J3 = 3
_K = 2
_DT = 0.025

_LR = _NG * _P


def _dot(a, b):
    return jnp.dot(a, b, preferred_element_type=jnp.float32)


def _xp(x2, w2):
    return x2[:, 0:1] * w2[0:1, :] + x2[:, 1:2] * w2[1:2, :]


_NB256 = 22 * _D
_XB256 = _NB256 + 15 * 8
_NB512 = 11 * _D
_XB512 = _NB512 + 3 * 8


def _seq(xr_ref, mr_ref, xt_ref, mt_ref, xl_ref, ml_ref,
         a1_ref, a2_ref, c3_ref, t256_ref, t512_ref,
         wout_ref, bout_ref, h0_ref,
         yr_ref, yt_ref, yl_ref):
    A1 = a1_ref[...]
    A2 = a2_ref[...]

    def mixA1(x):
        return (A1[:, 0:1] * x[0:1] + A1[:, 1:2] * x[1:2]
                + A1[:, 2:3] * x[2:3])

    def mixA2(ys):
        out = []
        for j in range(_J2):
            acc = A2[j:j + 1, 0:1] * ys[0]
            for q in range(1, _J2):
                acc = acc + A2[j:j + 1, q:q + 1] * ys[q]
            out.append(acc)
        return out

    def mixA3(ys):
        return [c3_ref[k, 0] * ys[0] + c3_ref[k, 1] * ys[1]
                + c3_ref[k, 2] * ys[2] for k in range(_J3)]

    W = lambda i: t256_ref[i * _D:(i + 1) * _D]
    Z = lambda i: t512_ref[i * _D:(i + 1) * _D]
    B = lambda i: t256_ref[_NB256 + 8 * i:_NB256 + 8 * i + 1]
    BZ = lambda i: t512_ref[_XB512 + 8 * i:_XB512 + 8 * i + 1]
    XW = lambda i: t256_ref[_XB256 + 8 * i:_XB256 + 8 * i + 2]
    XZ = lambda i: t512_ref[_NB512 + 8 * i:_NB512 + 8 * i + 2]

    wout = wout_ref[...]
    bout = bout_ref[...]

    def body(t, carry):
        hr = carry[0]
        htj = list(carry[1:1 + _J2])
        hlk = list(carry[1 + _J2:])

        xr = xr_ref[t] * mr_ref[t]
        xtv, mtv = xt_ref[t], mt_ref[t]
        xts = [xtv[j] * mtv[j] for j in range(_J2)]
        xlv, mlv = xl_ref[t], ml_ref[t]
        xls = [xlv[k] * mlv[k] for k in range(_J3)]

        g = jnp.tanh(mixA1(_dot(hr, W(0))) + B(0))
        g = jnp.tanh(mixA1(_dot(g, W(1))) + B(1))
        dh = mixA1(_dot(g, W(2))) + B(2)
        hr1 = hr + dh * (2.0 * _K * _DT)

        par2 = _dot(hr1, W(3))
        for _ in range(_K):
            y = mixA2([_dot(htj[j], W(4)) for j in range(_J2)])
            u = [jnp.tanh(par2 + y[j] + B(3)) for j in range(_J2)]
            y = mixA2([_dot(u[j], W(5)) for j in range(_J2)])
            u = [jnp.tanh(y[j] + B(4)) for j in range(_J2)]
            y = mixA2([_dot(u[j], W(6)) for j in range(_J2)])
            htj = [htj[j] + (y[j] + B(5)) * (_K * _DT) for j in range(_J2)]
            tcat = jnp.concatenate(htj[1:], axis=0)
            par3 = _dot(tcat, W(7))
            for _ in range(_K):
                y = mixA3([_dot(hlk[k], W(8)) for k in range(_J3)])
                v = [jnp.tanh(par3 + y[k] + B(6)) for k in range(_J3)]
                y = mixA3([_dot(v[k], W(9)) for k in range(_J3)])
                v = [jnp.tanh(y[k] + B(7)) for k in range(_J3)]
                y = mixA3([_dot(v[k], W(10)) for k in range(_J3)])
                hlk = [hlk[k] + (y[k] + B(8)) * _DT for k in range(_J3)]

        zr = jax.nn.sigmoid(mixA1(_xp(xr, XZ(0)) + _dot(hr1, Z(0))) + BZ(0))
        z, r = zr[:, :_D], zr[:, _D:]
        hh = jnp.tanh(mixA1(_xp(xr, XW(0)) + _dot(r * hr1, W(11))) + B(9))
        h21 = z * hr1 + (1.0 - z) * hh
        zr = jax.nn.sigmoid(mixA1(_dot(h21, Z(1)) + _dot(hr1, Z(2))) + BZ(1))
        z, r = zr[:, :_D], zr[:, _D:]
        hh = jnp.tanh(mixA1(_dot(h21, W(12)) + _dot(r * hr1, W(13))) + B(10))
        h21 = z * hr1 + (1.0 - z) * hh

        parz = _dot(h21, Z(3))
        y = mixA2([_dot(htj[j], Z(4)) + _xp(xts[j], XZ(1))
                   for j in range(_J2)])
        zrs = [jax.nn.sigmoid(parz + y[j] + BZ(2)) for j in range(_J2)]
        parh = _dot(h21, W(14))
        y = mixA2([_dot(zrs[j][:, _D:] * htj[j], W(15)) + _xp(xts[j], XW(1))
                   for j in range(_J2)])
        h22 = [zrs[j][:, :_D] * htj[j]
               + (1.0 - zrs[j][:, :_D]) * jnp.tanh(parh + y[j] + B(11))
               for j in range(_J2)]
        y = mixA2([_dot(h22[j], Z(5)) + _dot(htj[j], Z(6)) for j in range(_J2)])
        zrs = [jax.nn.sigmoid(y[j] + BZ(3)) for j in range(_J2)]
        y = mixA2([_dot(h22[j], W(16)) + _dot(zrs[j][:, _D:] * htj[j], W(17))
                   for j in range(_J2)])
        h22 = [zrs[j][:, :_D] * htj[j]
               + (1.0 - zrs[j][:, :_D]) * jnp.tanh(y[j] + B(12))
               for j in range(_J2)]

        t22 = jnp.concatenate(h22[1:], axis=0)
        parz = _dot(t22, Z(7))
        y = mixA3([_dot(hlk[k], Z(8)) + _xp(xls[k], XZ(2))
                   for k in range(_J3)])
        zrs = [jax.nn.sigmoid(parz + y[k] + BZ(4)) for k in range(_J3)]
        parh = _dot(t22, W(18))
        y = mixA3([_dot(zrs[k][:, _D:] * hlk[k], W(19)) + _xp(xls[k], XW(2))
                   for k in range(_J3)])
        h23 = [zrs[k][:, :_D] * hlk[k]
               + (1.0 - zrs[k][:, :_D]) * jnp.tanh(parh + y[k] + B(13))
               for k in range(_J3)]
        y = mixA3([_dot(h23[k], Z(9)) + _dot(hlk[k], Z(10)) for k in range(_J3)])
        zrs = [jax.nn.sigmoid(y[k] + BZ(5)) for k in range(_J3)]
        y = mixA3([_dot(h23[k], W(20)) + _dot(zrs[k][:, _D:] * hlk[k], W(21))
                   for k in range(_J3)])
        h23 = [zrs[k][:, :_D] * hlk[k]
               + (1.0 - zrs[k][:, :_D]) * jnp.tanh(y[k] + B(14))
               for k in range(_J3)]

        yr_ref[t] = _dot(h21, wout) + bout
        yt_ref[t] = _dot(jnp.concatenate(h22, axis=0), wout) + bout
        for k in range(_J3):
            yl_ref[t, k] = _dot(h23[k], wout) + bout
        return (h21, *h22, *h23)

    h0 = h0_ref[...]
    init = (jnp.broadcast_to(h0, (_P, _D)),
            *[jnp.broadcast_to(h0, (_P, _D)) for _ in range(_J2)],
            *[jnp.broadcast_to(h0, (_LR, _D)) for _ in range(_J3)])
    jax.lax.fori_loop(0, _T, body, init)


def kernel(t, mask, x2d, g1, g2, g3, adj1, adj2, adj3, d_adj, params):
    f32 = jnp.float32

    A1 = adj1[0]
    A2 = adj2[0]
    C3 = jnp.reshape(
        jnp.broadcast_to(jnp.transpose(adj3[0], (1, 2, 0))[:, :, :, None, None],
                         (_J3, _J3, _NG, _P, 1)),
        (_J3, _J3, _LR, 1))

    xj = jnp.transpose(x2d[0], (0, 2, 1, 3))
    mj = jnp.transpose(mask[0], (0, 2, 1, 3))
    xr = xj[:, 0]
    xt = xj[:, 1:7]
    xl = jnp.transpose(xj[:, 7:22].reshape(_T, _NG, _J3, _P, 2),
                       (0, 2, 1, 3, 4)).reshape(_T, _J3, _LR, 2)
    mr = mj[:, 0]
    mt = mj[:, 1:7]
    ml = jnp.transpose(mj[:, 7:22].reshape(_T, _NG, _J3, _P, 1),
                       (0, 2, 1, 3, 4)).reshape(_T, _J3, _LR, 1)

    def gru_split(cell, segs):
        Wz, Wr, Wh = cell["Wz"], cell["Wr"], cell["Wh"]
        out_zr, out_h = [], []
        o = 0
        for s in segs:
            out_zr.append(jnp.concatenate([Wz[o:o + s], Wr[o:o + s]], axis=1))
            out_h.append(Wh[o:o + s])
            o += s
        bzr = jnp.concatenate([cell["bz"], cell["br"]]).reshape(1, 2 * _D)
        bh = cell["bh"].reshape(1, _D)
        return out_zr, out_h, bzr, bh

    p = params
    o1 = p["ODE1"]; o2 = p["ODE2"]; o3 = p["ODE3"]
    g1c0_zr, g1c0_h, g1c0_bzr, g1c0_bh = gru_split(p["GRU1"][0], [2, _D])
    g1c1_zr, g1c1_h, g1c1_bzr, g1c1_bh = gru_split(p["GRU1"][1], [_D, _D])
    g2c0_zr, g2c0_h, g2c0_bzr, g2c0_bh = gru_split(p["GRU2"][0], [_D, 2, _D])
    g2c1_zr, g2c1_h, g2c1_bzr, g2c1_bh = gru_split(p["GRU2"][1], [_D, _D])
    g3c0_zr, g3c0_h, g3c0_bzr, g3c0_bh = gru_split(p["GRU3"][0], [_D, 2, _D])
    g3c1_zr, g3c1_h, g3c1_bzr, g3c1_bh = gru_split(p["GRU3"][1], [_D, _D])

    def pad8(v):
        return jnp.broadcast_to(v[:1], (8, v.shape[1])) if v.shape[0] == 1 \
            else jnp.concatenate([v, jnp.broadcast_to(v[1:2], (6, v.shape[1]))])

    t256 = jnp.concatenate([
        o1["W"][0], o1["W"][1], o1["W"][2],
        o2["W"][0][:_D], o2["W"][0][_D:], o2["W"][1], o2["W"][2],
        o3["W"][0][:_D], o3["W"][0][_D:], o3["W"][1], o3["W"][2],
        g1c0_h[1],
        g1c1_h[0], g1c1_h[1],
        g2c0_h[0], g2c0_h[2],
        g2c1_h[0], g2c1_h[1],
        g3c0_h[0], g3c0_h[2],
        g3c1_h[0], g3c1_h[1],
        pad8(o1["b"][0].reshape(1, _D)), pad8(o1["b"][1].reshape(1, _D)),
        pad8(o1["b"][2].reshape(1, _D)),
        pad8(o2["b"][0].reshape(1, _D)), pad8(o2["b"][1].reshape(1, _D)),
        pad8(o2["b"][2].reshape(1, _D)),
        pad8(o3["b"][0].reshape(1, _D)), pad8(o3["b"][1].reshape(1, _D)),
        pad8(o3["b"][2].reshape(1, _D)),
        pad8(g1c0_bh), pad8(g1c1_bh), pad8(g2c0_bh),
        pad8(g2c1_bh), pad8(g3c0_bh), pad8(g3c1_bh),
        pad8(g1c0_h[0]), pad8(g2c0_h[1]), pad8(g3c0_h[1]),
    ], axis=0)
    t512 = jnp.concatenate([
        g1c0_zr[1],
        g1c1_zr[0], g1c1_zr[1],
        g2c0_zr[0], g2c0_zr[2],
        g2c1_zr[0], g2c1_zr[1],
        g3c0_zr[0], g3c0_zr[2],
        g3c1_zr[0], g3c1_zr[1],
        pad8(g1c0_zr[0]), pad8(g2c0_zr[1]), pad8(g3c0_zr[1]),
        pad8(g1c0_bzr), pad8(g1c1_bzr), pad8(g2c0_bzr),
        pad8(g2c1_bzr), pad8(g3c0_bzr), pad8(g3c1_bzr),
    ], axis=0)

    wout = p["Wout"]
    bout = p["bout"].reshape(1, 3)
    h0 = p["h0"].reshape(1, _D)

    yr, yt_o, yl_o = pl.pallas_call(
        _seq,
        out_shape=[
            jax.ShapeDtypeStruct((_T, _P, 3), f32),
            jax.ShapeDtypeStruct((_T, _J2 * _P, 3), f32),
            jax.ShapeDtypeStruct((_T, _J3, _LR, 3), f32),
        ],
    )(xr, mr, xt, mt, xl, ml, A1, A2, C3, t256, t512, wout, bout, h0)

    yl = jnp.transpose(yl_o.reshape(_T, _J3, _NG, _P, 3),
                       (0, 2, 1, 3, 4)).reshape(_T, _NG * _J3 * _P, 3)
    y = jnp.concatenate([
        yr.reshape(_T, 1, _P, 3),
        yt_o.reshape(_T, _J2, _P, 3),
        yl.reshape(_T, _NG * _J3, _P, 3),
    ], axis=1)
    return jnp.transpose(y, (0, 2, 1, 3))[None]


# speedup vs baseline: 1.4199x; 1.4199x over previous
import jax
import jax.numpy as jnp
from jax.experimental import pallas as pl
from jax.experimental.pallas import tpu as pltpu

_T = 32
_P = 3
_D = 256
_NG = 5
_J2 = 6
_J3 = 3
_K = 2
_DT = 0.025

_LR = _NG * _P


def _dot(a, b):
    return jnp.dot(a, b, preferred_element_type=jnp.float32)


def _xp(x2, w2):
    return x2[:, 0:1] * w2[0:1, :] + x2[:, 1:2] * w2[1:2, :]


def _seq(xr_ref, mr_ref, xt_ref, mt_ref, xl_ref, ml_ref,
         a1_ref, a2_ref, c3_ref,
         o1w0, o1w1, o1w2, o1b0, o1b1, o1b2,
         o2w0, o2w1, o2w2, o2b0, o2b1, o2b2,
         o3w0, o3w1, o3w2, o3b0, o3b1, o3b2,
         g1c0z, g1c0r, g1c0h, g1c0bz, g1c0br, g1c0bh,
         g1c1z, g1c1r, g1c1h, g1c1bz, g1c1br, g1c1bh,
         g2c0z, g2c0r, g2c0h, g2c0bz, g2c0br, g2c0bh,
         g2c1z, g2c1r, g2c1h, g2c1bz, g2c1br, g2c1bh,
         g3c0z, g3c0r, g3c0h, g3c0bz, g3c0br, g3c0bh,
         g3c1z, g3c1r, g3c1h, g3c1bz, g3c1br, g3c1bh,
         wout_ref, bout_ref, h0_ref,
         yr_ref, yt_ref, yl_ref, al_s):
    A1 = a1_ref[...]
    A2 = a2_ref[...]

    def mixA1(x):
        return (A1[:, 0:1] * x[0:1] + A1[:, 1:2] * x[1:2]
                + A1[:, 2:3] * x[2:3])

    def mixA2(ys):
        out = []
        for j in range(_J2):
            acc = A2[j:j + 1, 0:1] * ys[0]
            for q in range(1, _J2):
                acc = acc + A2[j:j + 1, q:q + 1] * ys[q]
            out.append(acc)
        return out

    def mixA3(ys):
        return [c3_ref[k, 0] * ys[0] + c3_ref[k, 1] * ys[1]
                + c3_ref[k, 2] * ys[2] for k in range(_J3)]

    al_s[0 * _D:1 * _D] = g1c0z[2:2 + _D]
    al_s[1 * _D:2 * _D] = g1c0r[2:2 + _D]
    al_s[2 * _D:3 * _D] = g1c0h[2:2 + _D]
    al_s[3 * _D:4 * _D] = g2c0z[_D + 2:2 * _D + 2]
    al_s[4 * _D:5 * _D] = g2c0r[_D + 2:2 * _D + 2]
    al_s[5 * _D:6 * _D] = g2c0h[_D + 2:2 * _D + 2]
    al_s[6 * _D:7 * _D] = g3c0z[_D + 2:2 * _D + 2]
    al_s[7 * _D:8 * _D] = g3c0r[_D + 2:2 * _D + 2]
    al_s[8 * _D:9 * _D] = g3c0h[_D + 2:2 * _D + 2]
    AL = lambda i: al_s[i * _D:(i + 1) * _D]

    wout = wout_ref[...]
    bout = bout_ref[...]

    def body(t, carry):
        hr = carry[0]
        htj = list(carry[1:1 + _J2])
        hlk = list(carry[1 + _J2:])

        xr = xr_ref[t] * mr_ref[t]
        xtv, mtv = xt_ref[t], mt_ref[t]
        xts = [xtv[j] * mtv[j] for j in range(_J2)]
        xlv, mlv = xl_ref[t], ml_ref[t]
        xls = [xlv[k] * mlv[k] for k in range(_J3)]

        g = jnp.tanh(mixA1(_dot(hr, o1w0[...])) + o1b0[...])
        g = jnp.tanh(mixA1(_dot(g, o1w1[...])) + o1b1[...])
        dh = mixA1(_dot(g, o1w2[...])) + o1b2[...]
        hr1 = hr + dh * (2.0 * _K * _DT)

        par2 = _dot(hr1, o2w0[:_D])
        for _ in range(_K):
            y = mixA2([_dot(htj[j], o2w0[_D:]) for j in range(_J2)])
            u = [jnp.tanh(par2 + y[j] + o2b0[...]) for j in range(_J2)]
            y = mixA2([_dot(u[j], o2w1[...]) for j in range(_J2)])
            u = [jnp.tanh(y[j] + o2b1[...]) for j in range(_J2)]
            y = mixA2([_dot(u[j], o2w2[...]) for j in range(_J2)])
            htj = [htj[j] + (y[j] + o2b2[...]) * (_K * _DT) for j in range(_J2)]
            tcat = jnp.concatenate(htj[1:], axis=0)
            par3 = _dot(tcat, o3w0[:_D])
            for _ in range(_K):
                y = mixA3([_dot(hlk[k], o3w0[_D:]) for k in range(_J3)])
                v = [jnp.tanh(par3 + y[k] + o3b0[...]) for k in range(_J3)]
                y = mixA3([_dot(v[k], o3w1[...]) for k in range(_J3)])
                v = [jnp.tanh(y[k] + o3b1[...]) for k in range(_J3)]
                y = mixA3([_dot(v[k], o3w2[...]) for k in range(_J3)])
                hlk = [hlk[k] + (y[k] + o3b2[...]) * _DT for k in range(_J3)]

        z = jax.nn.sigmoid(mixA1(_xp(xr, g1c0z[0:2]) + _dot(hr1, AL(0))) + g1c0bz[...])
        r = jax.nn.sigmoid(mixA1(_xp(xr, g1c0r[0:2]) + _dot(hr1, AL(1))) + g1c0br[...])
        hh = jnp.tanh(mixA1(_xp(xr, g1c0h[0:2]) + _dot(r * hr1, AL(2))) + g1c0bh[...])
        h21 = z * hr1 + (1.0 - z) * hh
        z = jax.nn.sigmoid(mixA1(_dot(h21, g1c1z[:_D]) + _dot(hr1, g1c1z[_D:])) + g1c1bz[...])
        r = jax.nn.sigmoid(mixA1(_dot(h21, g1c1r[:_D]) + _dot(hr1, g1c1r[_D:])) + g1c1br[...])
        hh = jnp.tanh(mixA1(_dot(h21, g1c1h[:_D]) + _dot(r * hr1, g1c1h[_D:])) + g1c1bh[...])
        h21 = z * hr1 + (1.0 - z) * hh

        pz = _dot(h21, g2c0z[:_D])
        pr = _dot(h21, g2c0r[:_D])
        ph = _dot(h21, g2c0h[:_D])
        yz = mixA2([_dot(htj[j], AL(3)) + _xp(xts[j], g2c0z[_D:_D + 2])
                    for j in range(_J2)])
        yr_ = mixA2([_dot(htj[j], AL(4)) + _xp(xts[j], g2c0r[_D:_D + 2])
                     for j in range(_J2)])
        zs = [jax.nn.sigmoid(pz + yz[j] + g2c0bz[...]) for j in range(_J2)]
        rs = [jax.nn.sigmoid(pr + yr_[j] + g2c0br[...]) for j in range(_J2)]
        yh = mixA2([_dot(rs[j] * htj[j], AL(5)) + _xp(xts[j], g2c0h[_D:_D + 2])
                    for j in range(_J2)])
        h22 = [zs[j] * htj[j]
               + (1.0 - zs[j]) * jnp.tanh(ph + yh[j] + g2c0bh[...])
               for j in range(_J2)]
        yz = mixA2([_dot(h22[j], g2c1z[:_D]) + _dot(htj[j], g2c1z[_D:])
                    for j in range(_J2)])
        yr_ = mixA2([_dot(h22[j], g2c1r[:_D]) + _dot(htj[j], g2c1r[_D:])
                     for j in range(_J2)])
        zs = [jax.nn.sigmoid(yz[j] + g2c1bz[...]) for j in range(_J2)]
        rs = [jax.nn.sigmoid(yr_[j] + g2c1br[...]) for j in range(_J2)]
        yh = mixA2([_dot(h22[j], g2c1h[:_D]) + _dot(rs[j] * htj[j], g2c1h[_D:])
                    for j in range(_J2)])
        h22 = [zs[j] * htj[j]
               + (1.0 - zs[j]) * jnp.tanh(yh[j] + g2c1bh[...])
               for j in range(_J2)]

        t22 = jnp.concatenate(h22[1:], axis=0)
        pz = _dot(t22, g3c0z[:_D])
        pr = _dot(t22, g3c0r[:_D])
        ph = _dot(t22, g3c0h[:_D])
        yz = mixA3([_dot(hlk[k], AL(6)) + _xp(xls[k], g3c0z[_D:_D + 2])
                    for k in range(_J3)])
        yr_ = mixA3([_dot(hlk[k], AL(7)) + _xp(xls[k], g3c0r[_D:_D + 2])
                     for k in range(_J3)])
        zs = [jax.nn.sigmoid(pz + yz[k] + g3c0bz[...]) for k in range(_J3)]
        rs = [jax.nn.sigmoid(pr + yr_[k] + g3c0br[...]) for k in range(_J3)]
        yh = mixA3([_dot(rs[k] * hlk[k], AL(8)) + _xp(xls[k], g3c0h[_D:_D + 2])
                    for k in range(_J3)])
        h23 = [zs[k] * hlk[k]
               + (1.0 - zs[k]) * jnp.tanh(ph + yh[k] + g3c0bh[...])
               for k in range(_J3)]
        yz = mixA3([_dot(h23[k], g3c1z[:_D]) + _dot(hlk[k], g3c1z[_D:])
                    for k in range(_J3)])
        yr_ = mixA3([_dot(h23[k], g3c1r[:_D]) + _dot(hlk[k], g3c1r[_D:])
                     for k in range(_J3)])
        zs = [jax.nn.sigmoid(yz[k] + g3c1bz[...]) for k in range(_J3)]
        rs = [jax.nn.sigmoid(yr_[k] + g3c1br[...]) for k in range(_J3)]
        yh = mixA3([_dot(h23[k], g3c1h[:_D]) + _dot(rs[k] * hlk[k], g3c1h[_D:])
                    for k in range(_J3)])
        h23 = [zs[k] * hlk[k]
               + (1.0 - zs[k]) * jnp.tanh(yh[k] + g3c1bh[...])
               for k in range(_J3)]

        yr_ref[t] = _dot(h21, wout) + bout
        yt_ref[t] = _dot(jnp.concatenate(h22, axis=0), wout) + bout
        for k in range(_J3):
            yl_ref[t, k] = _dot(h23[k], wout) + bout
        return (h21, *h22, *h23)

    h0 = h0_ref[...]
    init = (jnp.broadcast_to(h0, (_P, _D)),
            *[jnp.broadcast_to(h0, (_P, _D)) for _ in range(_J2)],
            *[jnp.broadcast_to(h0, (_LR, _D)) for _ in range(_J3)])
    jax.lax.fori_loop(0, _T, body, init)


def kernel(t, mask, x2d, g1, g2, g3, adj1, adj2, adj3, d_adj, params):
    f32 = jnp.float32

    A1 = adj1[0]
    A2 = adj2[0]
    C3 = jnp.reshape(
        jnp.broadcast_to(jnp.transpose(adj3[0], (1, 2, 0))[:, :, :, None, None],
                         (_J3, _J3, _NG, _P, 1)),
        (_J3, _J3, _LR, 1))

    xj = jnp.transpose(x2d[0], (0, 2, 1, 3))
    mj = jnp.transpose(mask[0], (0, 2, 1, 3))
    xr = xj[:, 0]
    xt = xj[:, 1:7]
    xl = jnp.transpose(xj[:, 7:22].reshape(_T, _NG, _J3, _P, 2),
                       (0, 2, 1, 3, 4)).reshape(_T, _J3, _LR, 2)
    mr = mj[:, 0]
    mt = mj[:, 1:7]
    ml = jnp.transpose(mj[:, 7:22].reshape(_T, _NG, _J3, _P, 1),
                       (0, 2, 1, 3, 4)).reshape(_T, _J3, _LR, 1)

    p = params
    o1 = p["ODE1"]; o2 = p["ODE2"]; o3 = p["ODE3"]

    def b2(v):
        return v.reshape(1, -1)

    def cell_args(c):
        return (c["Wz"], c["Wr"], c["Wh"], b2(c["bz"]), b2(c["br"]), b2(c["bh"]))

    args = [xr, mr, xt, mt, xl, ml, A1, A2, C3,
            o1["W"][0], o1["W"][1], o1["W"][2],
            b2(o1["b"][0]), b2(o1["b"][1]), b2(o1["b"][2]),
            o2["W"][0], o2["W"][1], o2["W"][2],
            b2(o2["b"][0]), b2(o2["b"][1]), b2(o2["b"][2]),
            o3["W"][0], o3["W"][1], o3["W"][2],
            b2(o3["b"][0]), b2(o3["b"][1]), b2(o3["b"][2]),
            *cell_args(p["GRU1"][0]), *cell_args(p["GRU1"][1]),
            *cell_args(p["GRU2"][0]), *cell_args(p["GRU2"][1]),
            *cell_args(p["GRU3"][0]), *cell_args(p["GRU3"][1]),
            p["Wout"], b2(p["bout"]), p["h0"].reshape(1, _D)]

    yr, yt_o, yl_o = pl.pallas_call(
        _seq,
        out_shape=[
            jax.ShapeDtypeStruct((_T, _P, 3), f32),
            jax.ShapeDtypeStruct((_T, _J2 * _P, 3), f32),
            jax.ShapeDtypeStruct((_T, _J3, _LR, 3), f32),
        ],
        scratch_shapes=[pltpu.VMEM((9 * _D, _D), f32)],
    )(*args)

    yl = jnp.transpose(yl_o.reshape(_T, _J3, _NG, _P, 3),
                       (0, 2, 1, 3, 4)).reshape(_T, _NG * _J3 * _P, 3)
    y = jnp.concatenate([
        yr.reshape(_T, 1, _P, 3),
        yt_o.reshape(_T, _J2, _P, 3),
        yl.reshape(_T, _NG * _J3, _P, 3),
    ], axis=1)
    return jnp.transpose(y, (0, 2, 1, 3))[None]


# speedup vs baseline: 1.4492x; 1.0207x over previous
import jax
import jax.numpy as jnp
from jax.experimental import pallas as pl
from jax.experimental.pallas import tpu as pltpu

_T = 32
_P = 3
_D = 256
_NG = 5
_J2 = 6
_J3 = 3
_K = 2
_DT = 0.025

_LR = _NG * _P


def _dot(a, b):
    return jnp.dot(a, b, preferred_element_type=jnp.float32)


def _xp(x2, w2):
    return x2[:, 0:1] * w2[0:1, :] + x2[:, 1:2] * w2[1:2, :]


def _seq(x_ref, m_ref,
         a1_ref, a2_ref, c3_ref,
         o1w0, o1w1, o1w2, o1b0, o1b1, o1b2,
         o2w0, o2w1, o2w2, o2b0, o2b1, o2b2,
         o3w0, o3w1, o3w2, o3b0, o3b1, o3b2,
         g1c0z, g1c0r, g1c0h, g1c0bz, g1c0br, g1c0bh,
         g1c1z, g1c1r, g1c1h, g1c1bz, g1c1br, g1c1bh,
         g2c0z, g2c0r, g2c0h, g2c0bz, g2c0br, g2c0bh,
         g2c1z, g2c1r, g2c1h, g2c1bz, g2c1br, g2c1bh,
         g3c0z, g3c0r, g3c0h, g3c0bz, g3c0br, g3c0bh,
         g3c1z, g3c1r, g3c1h, g3c1bz, g3c1br, g3c1bh,
         wout_ref, bout_ref, h0_ref,
         yr_ref, yt_ref, yl_ref, al_s):
    A1 = a1_ref[...]
    A2 = a2_ref[...]

    def mixA1(x):
        return (A1[:, 0:1] * x[0:1] + A1[:, 1:2] * x[1:2]
                + A1[:, 2:3] * x[2:3])

    def mixA2(ys):
        out = []
        for j in range(_J2):
            acc = A2[j:j + 1, 0:1] * ys[0]
            for q in range(1, _J2):
                acc = acc + A2[j:j + 1, q:q + 1] * ys[q]
            out.append(acc)
        return out

    def mixA3(ys):
        return [c3_ref[k, 0] * ys[0] + c3_ref[k, 1] * ys[1]
                + c3_ref[k, 2] * ys[2] for k in range(_J3)]

    al_s[0 * _D:1 * _D] = g1c0z[2:2 + _D]
    al_s[1 * _D:2 * _D] = g1c0r[2:2 + _D]
    al_s[2 * _D:3 * _D] = g1c0h[2:2 + _D]
    al_s[3 * _D:4 * _D] = g2c0z[_D + 2:2 * _D + 2]
    al_s[4 * _D:5 * _D] = g2c0r[_D + 2:2 * _D + 2]
    al_s[5 * _D:6 * _D] = g2c0h[_D + 2:2 * _D + 2]
    al_s[6 * _D:7 * _D] = g3c0z[_D + 2:2 * _D + 2]
    al_s[7 * _D:8 * _D] = g3c0r[_D + 2:2 * _D + 2]
    al_s[8 * _D:9 * _D] = g3c0h[_D + 2:2 * _D + 2]
    AL = lambda i: al_s[i * _D:(i + 1) * _D]

    wout = wout_ref[...]
    bout = bout_ref[...]

    def body(t, carry):
        hr = carry[0]
        htj = list(carry[1:1 + _J2])
        hlk = list(carry[1 + _J2:])

        xv = x_ref[t] * m_ref[t]
        xr = xv[:, 0]
        xts = [xv[:, 1 + j] for j in range(_J2)]
        xls = [jnp.concatenate([xv[:, 7 + 3 * g + k] for g in range(_NG)],
                               axis=0) for k in range(_J3)]

        g = jnp.tanh(mixA1(_dot(hr, o1w0[...])) + o1b0[...])
        g = jnp.tanh(mixA1(_dot(g, o1w1[...])) + o1b1[...])
        dh = mixA1(_dot(g, o1w2[...])) + o1b2[...]
        hr1 = hr + dh * (2.0 * _K * _DT)

        par2 = _dot(hr1, o2w0[:_D])
        for _ in range(_K):
            y = mixA2([_dot(htj[j], o2w0[_D:]) for j in range(_J2)])
            u = [jnp.tanh(par2 + y[j] + o2b0[...]) for j in range(_J2)]
            y = mixA2([_dot(u[j], o2w1[...]) for j in range(_J2)])
            u = [jnp.tanh(y[j] + o2b1[...]) for j in range(_J2)]
            y = mixA2([_dot(u[j], o2w2[...]) for j in range(_J2)])
            htj = [htj[j] + (y[j] + o2b2[...]) * (_K * _DT) for j in range(_J2)]
            tcat = jnp.concatenate(htj[1:], axis=0)
            par3 = _dot(tcat, o3w0[:_D])
            for _ in range(_K):
                y = mixA3([_dot(hlk[k], o3w0[_D:]) for k in range(_J3)])
                v = [jnp.tanh(par3 + y[k] + o3b0[...]) for k in range(_J3)]
                y = mixA3([_dot(v[k], o3w1[...]) for k in range(_J3)])
                v = [jnp.tanh(y[k] + o3b1[...]) for k in range(_J3)]
                y = mixA3([_dot(v[k], o3w2[...]) for k in range(_J3)])
                hlk = [hlk[k] + (y[k] + o3b2[...]) * _DT for k in range(_J3)]

        z = jax.nn.sigmoid(mixA1(_xp(xr, g1c0z[0:2]) + _dot(hr1, AL(0))) + g1c0bz[...])
        r = jax.nn.sigmoid(mixA1(_xp(xr, g1c0r[0:2]) + _dot(hr1, AL(1))) + g1c0br[...])
        hh = jnp.tanh(mixA1(_xp(xr, g1c0h[0:2]) + _dot(r * hr1, AL(2))) + g1c0bh[...])
        h21 = z * hr1 + (1.0 - z) * hh
        z = jax.nn.sigmoid(mixA1(_dot(h21, g1c1z[:_D]) + _dot(hr1, g1c1z[_D:])) + g1c1bz[...])
        r = jax.nn.sigmoid(mixA1(_dot(h21, g1c1r[:_D]) + _dot(hr1, g1c1r[_D:])) + g1c1br[...])
        hh = jnp.tanh(mixA1(_dot(h21, g1c1h[:_D]) + _dot(r * hr1, g1c1h[_D:])) + g1c1bh[...])
        h21 = z * hr1 + (1.0 - z) * hh

        pz = _dot(h21, g2c0z[:_D])
        pr = _dot(h21, g2c0r[:_D])
        ph = _dot(h21, g2c0h[:_D])
        yz = mixA2([_dot(htj[j], AL(3)) + _xp(xts[j], g2c0z[_D:_D + 2])
                    for j in range(_J2)])
        yr_ = mixA2([_dot(htj[j], AL(4)) + _xp(xts[j], g2c0r[_D:_D + 2])
                     for j in range(_J2)])
        zs = [jax.nn.sigmoid(pz + yz[j] + g2c0bz[...]) for j in range(_J2)]
        rs = [jax.nn.sigmoid(pr + yr_[j] + g2c0br[...]) for j in range(_J2)]
        yh = mixA2([_dot(rs[j] * htj[j], AL(5)) + _xp(xts[j], g2c0h[_D:_D + 2])
                    for j in range(_J2)])
        h22 = [zs[j] * htj[j]
               + (1.0 - zs[j]) * jnp.tanh(ph + yh[j] + g2c0bh[...])
               for j in range(_J2)]
        yz = mixA2([_dot(h22[j], g2c1z[:_D]) + _dot(htj[j], g2c1z[_D:])
                    for j in range(_J2)])
        yr_ = mixA2([_dot(h22[j], g2c1r[:_D]) + _dot(htj[j], g2c1r[_D:])
                     for j in range(_J2)])
        zs = [jax.nn.sigmoid(yz[j] + g2c1bz[...]) for j in range(_J2)]
        rs = [jax.nn.sigmoid(yr_[j] + g2c1br[...]) for j in range(_J2)]
        yh = mixA2([_dot(h22[j], g2c1h[:_D]) + _dot(rs[j] * htj[j], g2c1h[_D:])
                    for j in range(_J2)])
        h22 = [zs[j] * htj[j]
               + (1.0 - zs[j]) * jnp.tanh(yh[j] + g2c1bh[...])
               for j in range(_J2)]

        t22 = jnp.concatenate(h22[1:], axis=0)
        pz = _dot(t22, g3c0z[:_D])
        pr = _dot(t22, g3c0r[:_D])
        ph = _dot(t22, g3c0h[:_D])
        yz = mixA3([_dot(hlk[k], AL(6)) + _xp(xls[k], g3c0z[_D:_D + 2])
                    for k in range(_J3)])
        yr_ = mixA3([_dot(hlk[k], AL(7)) + _xp(xls[k], g3c0r[_D:_D + 2])
                     for k in range(_J3)])
        zs = [jax.nn.sigmoid(pz + yz[k] + g3c0bz[...]) for k in range(_J3)]
        rs = [jax.nn.sigmoid(pr + yr_[k] + g3c0br[...]) for k in range(_J3)]
        yh = mixA3([_dot(rs[k] * hlk[k], AL(8)) + _xp(xls[k], g3c0h[_D:_D + 2])
                    for k in range(_J3)])
        h23 = [zs[k] * hlk[k]
               + (1.0 - zs[k]) * jnp.tanh(ph + yh[k] + g3c0bh[...])
               for k in range(_J3)]
        yz = mixA3([_dot(h23[k], g3c1z[:_D]) + _dot(hlk[k], g3c1z[_D:])
                    for k in range(_J3)])
        yr_ = mixA3([_dot(h23[k], g3c1r[:_D]) + _dot(hlk[k], g3c1r[_D:])
                     for k in range(_J3)])
        zs = [jax.nn.sigmoid(yz[k] + g3c1bz[...]) for k in range(_J3)]
        rs = [jax.nn.sigmoid(yr_[k] + g3c1br[...]) for k in range(_J3)]
        yh = mixA3([_dot(h23[k], g3c1h[:_D]) + _dot(rs[k] * hlk[k], g3c1h[_D:])
                    for k in range(_J3)])
        h23 = [zs[k] * hlk[k]
               + (1.0 - zs[k]) * jnp.tanh(yh[k] + g3c1bh[...])
               for k in range(_J3)]

        yr_ref[t] = _dot(h21, wout) + bout
        yt_ref[t] = _dot(jnp.concatenate(h22, axis=0), wout) + bout
        for k in range(_J3):
            yl_ref[t, k] = _dot(h23[k], wout) + bout
        return (h21, *h22, *h23)

    h0 = h0_ref[...]
    init = (jnp.broadcast_to(h0, (_P, _D)),
            *[jnp.broadcast_to(h0, (_P, _D)) for _ in range(_J2)],
            *[jnp.broadcast_to(h0, (_LR, _D)) for _ in range(_J3)])
    jax.lax.fori_loop(0, _T, body, init)


def kernel(t, mask, x2d, g1, g2, g3, adj1, adj2, adj3, d_adj, params):
    f32 = jnp.float32

    A1 = adj1[0]
    A2 = adj2[0]
    C3 = jnp.reshape(
        jnp.broadcast_to(jnp.transpose(adj3[0], (1, 2, 0))[:, :, :, None, None],
                         (_J3, _J3, _NG, _P, 1)),
        (_J3, _J3, _LR, 1))

    p = params
    o1 = p["ODE1"]; o2 = p["ODE2"]; o3 = p["ODE3"]

    def b2(v):
        return v.reshape(1, -1)

    def cell_args(c):
        return (c["Wz"], c["Wr"], c["Wh"], b2(c["bz"]), b2(c["br"]), b2(c["bh"]))

    args = [x2d[0], mask[0], A1, A2, C3,
            o1["W"][0], o1["W"][1], o1["W"][2],
            b2(o1["b"][0]), b2(o1["b"][1]), b2(o1["b"][2]),
            o2["W"][0], o2["W"][1], o2["W"][2],
            b2(o2["b"][0]), b2(o2["b"][1]), b2(o2["b"][2]),
            o3["W"][0], o3["W"][1], o3["W"][2],
            b2(o3["b"][0]), b2(o3["b"][1]), b2(o3["b"][2]),
            *cell_args(p["GRU1"][0]), *cell_args(p["GRU1"][1]),
            *cell_args(p["GRU2"][0]), *cell_args(p["GRU2"][1]),
            *cell_args(p["GRU3"][0]), *cell_args(p["GRU3"][1]),
            p["Wout"], b2(p["bout"]), p["h0"].reshape(1, _D)]

    yr, yt_o, yl_o = pl.pallas_call(
        _seq,
        out_shape=[
            jax.ShapeDtypeStruct((_T, _P, 3), f32),
            jax.ShapeDtypeStruct((_T, _J2 * _P, 3), f32),
            jax.ShapeDtypeStruct((_T, _J3, _LR, 3), f32),
        ],
        scratch_shapes=[pltpu.VMEM((9 * _D, _D), f32)],
    )(*args)

    yl = jnp.transpose(yl_o.reshape(_T, _J3, _NG, _P, 3),
                       (0, 2, 1, 3, 4)).reshape(_T, _NG * _J3 * _P, 3)
    y = jnp.concatenate([
        yr.reshape(_T, 1, _P, 3),
        yt_o.reshape(_T, _J2, _P, 3),
        yl.reshape(_T, _NG * _J3, _P, 3),
    ], axis=1)
    return jnp.transpose(y, (0, 2, 1, 3))[None]
